# baseline (speedup 1.0000x reference)
import functools

import jax
import jax.numpy as jnp
from jax import lax
from jax.experimental import pallas as pl
from jax.experimental.pallas import tpu as pltpu
from jax.experimental.pallas import tpu_sc as plsc

_VOCAB = 1000000
_B = 16384
_EMB = 64
_PHYS = 1000064
_WPW = 245
_RANGE = _WPW * 128
_CW = 512
_NCH = 62
_LCAP = 1280
_CCAP = 128
_OUT_ROWS = 2 * _B + 1024
_DUMP = 2 * _B


def _sc_gather(tableT, tgt, ctx):
    info = plsc.get_sparse_core_info()
    nc, ns = info.num_cores, info.num_subcores
    mesh = plsc.VectorSubcoreMesh(core_axis_name="c", subcore_axis_name="s")

    @functools.partial(
        pl.kernel,
        mesh=mesh,
        compiler_params=pltpu.CompilerParams(needs_layout_passes=False),
        out_type=jax.ShapeDtypeStruct((_OUT_ROWS, 128), jnp.float32),
        scratch_types=[
            pltpu.VMEM((4096,), jnp.int32),
            pltpu.VMEM((_LCAP,), jnp.int32),
            pltpu.VMEM((_LCAP,), jnp.int32),
            pltpu.VMEM((_CCAP,), jnp.int32),
            pltpu.VMEM((_CCAP,), jnp.int32),
            pltpu.VMEM((_EMB, _CW), jnp.float32),
            pltpu.VMEM((_EMB, _CW), jnp.float32),
            pltpu.VMEM((_CCAP, 128), jnp.float32),
            pltpu.SemaphoreType.DMA,
            pltpu.SemaphoreType.DMA,
            pltpu.SemaphoreType.DMA,
        ],
    )
    def k(tgt_hbm, ctx_hbm, tT_hbm, out_hbm, seg_v, mid_v, mbp_v, ccol_v,
          cbp_v, buf_a, buf_b, scb_v, sem_a, sem_b, sem_s):
        wid = lax.axis_index("s") * nc + lax.axis_index("c")
        lo = wid * _RANGE
        hi = lo + _RANGE
        iota16 = lax.iota(jnp.int32, 16)
        zeros16 = jnp.zeros((16,), jnp.int32)
        for zg in range(_CCAP // 16):
            ccol_v[pl.ds(zg * 16, 16)] = zeros16
            cbp_v[pl.ds(zg * 16, 16)] = zeros16 + _DUMP

        def scan_seg(idx_hbm, seg_base, bpos_off, cnt):
            pltpu.sync_copy(idx_hbm.at[pl.ds(seg_base, 4096)], seg_v)

            def grp(g, cnt):
                v = seg_v[pl.ds(g * 16, 16)]
                mask = (v >= lo) & (v < hi)
                mcount = jnp.sum(mask.astype(jnp.int32))
                plsc.store_compressed(mid_v.at[pl.ds(cnt, 16)], v, mask=mask)
                bpos = iota16 + (bpos_off + seg_base + g * 16)
                plsc.store_compressed(mbp_v.at[pl.ds(cnt, 16)], bpos, mask=mask)
                return cnt + mcount

            return lax.fori_loop(0, 256, grp, cnt)

        cnt = jnp.int32(0)
        for s in range(4):
            cnt = scan_seg(tgt_hbm, s * 4096, 0, cnt)
        for s in range(4):
            cnt = scan_seg(ctx_hbm, s * 4096, _B, cnt)

        def dstart(c):
            return pl.multiple_of(
                jnp.minimum(lo + c * _CW, _PHYS - _CW), 128)

        bufs = ((buf_a, sem_a), (buf_b, sem_b))
        for b in range(2):
            pltpu.async_copy(
                tT_hbm.at[:, pl.ds(dstart(jnp.int32(b)), _CW)],
                bufs[b][0], bufs[b][1])

        n_lg = (cnt + 15) >> 4
        dump_vec = jnp.full((16,), _DUMP, jnp.int32)

        def do_chunk(c, buf, sem):
            pltpu.make_async_copy(
                tT_hbm.at[:, pl.ds(dstart(c), _CW)], buf, sem).wait()
            ds0 = dstart(c)

            def grp2(g, mcnt):
                vmask = (iota16 + g * 16) < cnt
                mv = mid_v[pl.ds(g * 16, 16)]
                bv = mbp_v[pl.ds(g * 16, 16)]
                inm = (mv >= ds0) & (mv < ds0 + _CW) & vmask
                m2 = jnp.sum(inm.astype(jnp.int32))
                plsc.store_compressed(
                    ccol_v.at[pl.ds(mcnt, 16)], mv - ds0, mask=inm)
                plsc.store_compressed(cbp_v.at[pl.ds(mcnt, 16)], bv, mask=inm)
                return mcnt + m2

            mcnt = lax.fori_loop(0, n_lg, grp2, jnp.int32(0))
            cbp_v[pl.ds(mcnt, 16)] = dump_vec

            def mgrp(mg, carry):
                colv = ccol_v[pl.ds(mg * 16, 16)]
                rowv = iota16 + mg * 16
                vm = rowv < mcnt
                for e in range(_EMB):
                    e_vec = jnp.full((16,), e, jnp.int32)
                    vals = plsc.load_gather(buf, [e_vec, colv], mask=vm)
                    plsc.store_scatter(scb_v, [rowv, e_vec], vals, mask=vm)
                return carry

            n_mg = (mcnt + 15) >> 4
            lax.fori_loop(0, n_mg, mgrp, jnp.int32(0))

            def scat(sg, carry):
                bvec = cbp_v[pl.ds(sg * 16, 16)]
                pltpu.async_copy(
                    scb_v.at[pl.ds(sg * 16, 16)], out_hbm.at[bvec], sem_s)
                return carry

            lax.fori_loop(0, n_mg, scat, jnp.int32(0))

            def drain(sg, carry):
                pltpu.make_async_copy(
                    out_hbm.at[pl.ds(0, 16)], scb_v.at[pl.ds(0, 16)],
                    sem_s).wait()
                return carry

            lax.fori_loop(0, n_mg, drain, jnp.int32(0))

        def superstep(ss, carry):
            for b in range(2):
                c = ss * 2 + b
                buf, sem = bufs[b]
                do_chunk(c, buf, sem)

                @pl.when(c + 2 < _NCH)
                def _():
                    pltpu.async_copy(
                        tT_hbm.at[:, pl.ds(dstart(c + 2), _CW)], buf, sem)

            return carry

        lax.fori_loop(0, _NCH // 2, superstep, jnp.int32(0))

    return k(tgt, ctx, tableT)


def _tc_mlp(xy, w1, b1, w2, b2, w3, b3):
    blk = 1024
    n_blk = _B // blk

    def body(x_ref, y_ref, w1_ref, b1_ref, w2_ref, b2_ref, w3_ref, b3_ref,
             o_ref):
        shared = x_ref[:, :_EMB] * y_ref[:, :_EMB]
        h1 = jnp.maximum(
            jnp.dot(shared, w1_ref[...], preferred_element_type=jnp.float32)
            + b1_ref[...], 0.0)
        h2 = jnp.maximum(
            jnp.dot(h1, w2_ref[...], preferred_element_type=jnp.float32)
            + b2_ref[...], 0.0)
        z = jnp.dot(h2, w3_ref[...], preferred_element_type=jnp.float32) \
            + b3_ref[...]
        o_ref[...] = jax.nn.sigmoid(z)

    zero2 = lambda i: (0, 0)
    return pl.pallas_call(
        body,
        grid=(n_blk,),
        in_specs=[
            pl.BlockSpec((blk, 128), lambda i: (i, 0)),
            pl.BlockSpec((blk, 128), lambda i: (i + n_blk, 0)),
            pl.BlockSpec((_EMB, 64), zero2),
            pl.BlockSpec((1, 64), zero2),
            pl.BlockSpec((64, 32), zero2),
            pl.BlockSpec((1, 32), zero2),
            pl.BlockSpec((32, 1), zero2),
            pl.BlockSpec((1, 1), zero2),
        ],
        out_specs=pl.BlockSpec((blk, 1), lambda i: (i, 0)),
        out_shape=jax.ShapeDtypeStruct((_B, 1), jnp.float32),
    )(xy, xy, w1, b1, w2, b2, w3, b3)


def kernel(target_word, context_word, table, W1, b1, W2, b2, W3, b3):
    xy = _sc_gather(table.T, target_word.astype(jnp.int32),
                    context_word.astype(jnp.int32))
    out = _tc_mlp(xy, W1, b1.reshape(1, -1), W2, b2.reshape(1, -1), W3,
                  b3.reshape(1, 1))
    return jnp.reshape(out, (-1,))
